# Initial kernel scaffold; baseline (speedup 1.0000x reference)
import functools

import jax
import jax.numpy as jnp
from jax import lax
from jax.experimental import pallas as pl
from jax.experimental.pallas import tpu as pltpu
from jax.experimental.pallas import tpu_sc as plsc

_LANES = 16


@functools.lru_cache(maxsize=None)
def _build(N, B, T, C):
    assert N % C == 0 and C % _LANES == 0 and C % 8 == 0
    NCH = N // C
    mesh = plsc.VectorSubcoreMesh(core_axis_name="c", subcore_axis_name="s")
    NC = mesh.num_cores
    NW = NC * mesh.num_subcores
    n_rounds = (NCH + NW - 1) // NW

    @functools.partial(
        pl.kernel,
        mesh=mesh,
        out_type=(
            jax.ShapeDtypeStruct((N,), jnp.float32),
            jax.ShapeDtypeStruct((N,), jnp.float32),
        ),
        scratch_types=[
            pltpu.VMEM((B,), jnp.int32),
            pltpu.VMEM((T,), jnp.float32),
            pltpu.VMEM((T,), jnp.float32),
            pltpu.VMEM((C,), jnp.int32),
            pltpu.VMEM((C,), jnp.float32),
            pltpu.VMEM((C,), jnp.float32),
        ],
    )
    def k(batch_h, time_h, ds_h, ns_h, outd_h, outn_h,
          time_v, ds_v, ns_v, idx_v, d_v, n_v):
        wid = lax.axis_index("s") * NC + lax.axis_index("c")
        pltpu.sync_copy(time_h, time_v)
        pltpu.sync_copy(ds_h, ds_v)
        pltpu.sync_copy(ns_h, ns_v)

        def do_chunk(c):
            base = c * C
            pltpu.sync_copy(batch_h.at[pl.ds(base, C)], idx_v)

            def body(j, carry):
                sl = pl.ds(j * _LANES, _LANES)
                idx = idx_v[sl]
                t = plsc.load_gather(time_v, [idx])
                d_v[sl] = plsc.load_gather(ds_v, [t])
                n_v[sl] = plsc.load_gather(ns_v, [t])
                return carry

            lax.fori_loop(0, C // _LANES, body, 0)
            pltpu.sync_copy(d_v, outd_h.at[pl.ds(base, C)])
            pltpu.sync_copy(n_v, outn_h.at[pl.ds(base, C)])

        def round_body(r, carry):
            c = wid + r * NW

            @pl.when(c < NCH)
            def _():
                do_chunk(c)

            return carry

        lax.fori_loop(0, n_rounds, round_body, 0)

    return k


def kernel(batch, time, forward_data_schedule, forward_noise_schedule):
    N = batch.shape[0]
    B = time.shape[0]
    T = forward_data_schedule.shape[0]
    C = 4000
    while N % C or C % _LANES:
        C //= 2
    d, n = _build(N, B, T, C)(
        batch, time, forward_data_schedule, forward_noise_schedule)
    return d[:, None], n[:, None]


# speedup vs baseline: 160.3785x; 160.3785x over previous
import functools

import jax
import jax.numpy as jnp
from jax import lax
from jax.experimental import pallas as pl
from jax.experimental.pallas import tpu as pltpu
from jax.experimental.pallas import tpu_sc as plsc

_LANES = 16


@functools.lru_cache(maxsize=None)
def _build(N, B, T, C):
    assert N % C == 0 and C % _LANES == 0 and C % 8 == 0
    NCH = N // C
    mesh = plsc.VectorSubcoreMesh(core_axis_name="c", subcore_axis_name="s")
    NC = mesh.num_cores
    NW = NC * mesh.num_subcores
    n_rounds = (NCH + NW - 1) // NW

    @functools.partial(
        pl.kernel,
        mesh=mesh,
        out_type=(
            jax.ShapeDtypeStruct((N,), jnp.float32),
            jax.ShapeDtypeStruct((N,), jnp.float32),
        ),
        scratch_types=[
            pltpu.VMEM((B,), jnp.int32),
            pltpu.VMEM((T,), jnp.float32),
            pltpu.VMEM((T,), jnp.float32),
            pltpu.VMEM((C,), jnp.int32),
            pltpu.VMEM((C,), jnp.float32),
            pltpu.VMEM((C,), jnp.float32),
        ],
        compiler_params=pltpu.CompilerParams(needs_layout_passes=False),
    )
    def k(batch_h, time_h, ds_h, ns_h, outd_h, outn_h,
          time_v, ds_v, ns_v, idx_v, d_v, n_v):
        wid = lax.axis_index("s") * NC + lax.axis_index("c")
        pltpu.sync_copy(time_h, time_v)
        pltpu.sync_copy(ds_h, ds_v)
        pltpu.sync_copy(ns_h, ns_v)

        def do_chunk(c):
            base = c * C
            pltpu.sync_copy(batch_h.at[pl.ds(base, C)], idx_v)

            def body(j, carry):
                sl = pl.ds(j * _LANES, _LANES)
                idx = idx_v[sl]
                t = plsc.load_gather(time_v, [idx])
                d_v[sl] = plsc.load_gather(ds_v, [t])
                n_v[sl] = plsc.load_gather(ns_v, [t])
                return carry

            lax.fori_loop(0, C // _LANES, body, 0)
            pltpu.sync_copy(d_v, outd_h.at[pl.ds(base, C)])
            pltpu.sync_copy(n_v, outn_h.at[pl.ds(base, C)])

        def round_body(r, carry):
            c = wid + r * NW

            @pl.when(c < NCH)
            def _():
                do_chunk(c)

            return carry

        lax.fori_loop(0, n_rounds, round_body, 0)

    return k


def kernel(batch, time, forward_data_schedule, forward_noise_schedule):
    N = batch.shape[0]
    B = time.shape[0]
    T = forward_data_schedule.shape[0]
    C = 4000
    while N % C or C % _LANES:
        C //= 2
    d, n = _build(N, B, T, C)(
        batch, time, forward_data_schedule, forward_noise_schedule)
    return d[:, None], n[:, None]


# speedup vs baseline: 232.0067x; 1.4466x over previous
import functools

import jax
import jax.numpy as jnp
from jax import lax
from jax.experimental import pallas as pl
from jax.experimental.pallas import tpu as pltpu
from jax.experimental.pallas import tpu_sc as plsc

_LANES = 16


@functools.lru_cache(maxsize=None)
def _build(N, B, T, C):
    assert N % C == 0 and C % _LANES == 0 and C % 8 == 0
    NCH = N // C
    mesh = plsc.VectorSubcoreMesh(core_axis_name="c", subcore_axis_name="s")
    NC = mesh.num_cores
    NW = NC * mesh.num_subcores
    FULL = NCH // NW
    TAIL = NCH - FULL * NW
    UNROLL = 10
    assert (C // _LANES) % UNROLL == 0

    @functools.partial(
        pl.kernel,
        mesh=mesh,
        out_type=(
            jax.ShapeDtypeStruct((N,), jnp.float32),
            jax.ShapeDtypeStruct((N,), jnp.float32),
        ),
        scratch_types=[
            pltpu.VMEM((B,), jnp.int32),
            pltpu.VMEM((T,), jnp.float32),
            pltpu.VMEM((T,), jnp.float32),
            [pltpu.VMEM((C,), jnp.int32) for _ in range(2)],
            [pltpu.VMEM((C,), jnp.float32) for _ in range(2)],
            [pltpu.VMEM((C,), jnp.float32) for _ in range(2)],
            [pltpu.SemaphoreType.DMA for _ in range(2)],
            [pltpu.SemaphoreType.DMA for _ in range(2)],
            [pltpu.SemaphoreType.DMA for _ in range(2)],
        ],
        compiler_params=pltpu.CompilerParams(needs_layout_passes=False),
    )
    def k(batch_h, time_h, ds_h, ns_h, outd_h, outn_h,
          time_v, ds_v, ns_v, idx_v, d_v, n_v, s_in, s_d, s_n):
        wid = lax.axis_index("s") * NC + lax.axis_index("c")

        def base_of(r):
            return (wid + r * NW) * C

        def compute(b):
            iv, dv, nv = idx_v[b], d_v[b], n_v[b]

            @plsc.parallel_loop(0, C, _LANES, unroll=UNROLL)
            def _(i):
                sl = pl.ds(i, _LANES)
                idx = iv[sl]
                t = plsc.load_gather(time_v, [idx])
                dv[sl] = plsc.load_gather(ds_v, [t])
                nv[sl] = plsc.load_gather(ns_v, [t])

        cp_in = [None, None]
        cp_d = [None, None]
        cp_n = [None, None]
        cp_in[0] = pltpu.async_copy(
            batch_h.at[pl.ds(base_of(0), C)], idx_v[0], s_in[0])
        pltpu.sync_copy(time_h, time_v)
        pltpu.sync_copy(ds_h, ds_v)
        pltpu.sync_copy(ns_h, ns_v)

        for r in range(FULL):
            b = r % 2
            if r + 1 < FULL:
                cp_in[1 - b] = pltpu.async_copy(
                    batch_h.at[pl.ds(base_of(r + 1), C)],
                    idx_v[1 - b], s_in[1 - b])
            cp_in[b].wait()
            if r >= 2:
                cp_d[b].wait()
                cp_n[b].wait()
            compute(b)
            cp_d[b] = pltpu.async_copy(
                d_v[b], outd_h.at[pl.ds(base_of(r), C)], s_d[b])
            cp_n[b] = pltpu.async_copy(
                n_v[b], outn_h.at[pl.ds(base_of(r), C)], s_n[b])

        for r in (FULL - 2, FULL - 1):
            if r >= 0:
                cp_d[r % 2].wait()
                cp_n[r % 2].wait()

        if TAIL:
            @pl.when(wid < TAIL)
            def _():
                base = (FULL * NW + wid) * C
                pltpu.sync_copy(batch_h.at[pl.ds(base, C)], idx_v[0])
                compute(0)
                pltpu.sync_copy(d_v[0], outd_h.at[pl.ds(base, C)])
                pltpu.sync_copy(n_v[0], outn_h.at[pl.ds(base, C)])

    return k


def kernel(batch, time, forward_data_schedule, forward_noise_schedule):
    N = batch.shape[0]
    B = time.shape[0]
    T = forward_data_schedule.shape[0]
    C = 4000
    while N % C or C % _LANES:
        C //= 2
    d, n = _build(N, B, T, C)(
        batch, time, forward_data_schedule, forward_noise_schedule)
    return d[:, None], n[:, None]


# speedup vs baseline: 267.1304x; 1.1514x over previous
import functools

import jax
import jax.numpy as jnp
from jax import lax
from jax.experimental import pallas as pl
from jax.experimental.pallas import tpu as pltpu
from jax.experimental.pallas import tpu_sc as plsc

_LANES = 16


@functools.lru_cache(maxsize=None)
def _build(N, B, T, C):
    assert N % C == 0 and C % _LANES == 0 and C % 8 == 0
    NCH = N // C
    mesh = plsc.VectorSubcoreMesh(core_axis_name="c", subcore_axis_name="s")
    NC = mesh.num_cores
    NW = NC * mesh.num_subcores
    FULL = NCH // NW
    TAIL = NCH - FULL * NW
    UNROLL = 10
    assert (C // _LANES) % UNROLL == 0

    @functools.partial(
        pl.kernel,
        mesh=mesh,
        out_type=jax.ShapeDtypeStruct((N,), jnp.float32),
        scratch_types=[
            pltpu.VMEM((B,), jnp.int32),
            pltpu.VMEM((T,), jnp.float32),
            [pltpu.VMEM((C,), jnp.int32) for _ in range(2)],
            [pltpu.VMEM((C,), jnp.float32) for _ in range(2)],
            [pltpu.SemaphoreType.DMA for _ in range(2)],
            [pltpu.SemaphoreType.DMA for _ in range(2)],
        ],
        compiler_params=pltpu.CompilerParams(
            needs_layout_passes=False, use_tc_tiling_on_sc=False),
    )
    def k(batch_h, time_h, sched_h, out_h, time_v, sched_v, idx_v, o_v,
          s_in, s_out):
        wid = lax.axis_index("s") * NC + lax.axis_index("c")

        def base_of(r):
            return (wid + r * NW) * C

        def compute(b):
            iv, ov = idx_v[b], o_v[b]

            @plsc.parallel_loop(0, C, _LANES, unroll=UNROLL)
            def _(i):
                sl = pl.ds(i, _LANES)
                idx = iv[sl]
                t = plsc.load_gather(time_v, [idx])
                ov[sl] = plsc.load_gather(sched_v, [t])

        cp_in = [None, None]
        cp_out = [None, None]
        cp_in[0] = pltpu.async_copy(
            batch_h.at[pl.ds(base_of(0), C)], idx_v[0], s_in[0])
        pltpu.sync_copy(time_h, time_v)
        pltpu.sync_copy(sched_h, sched_v)

        for r in range(FULL):
            b = r % 2
            if r + 1 < FULL:
                cp_in[1 - b] = pltpu.async_copy(
                    batch_h.at[pl.ds(base_of(r + 1), C)],
                    idx_v[1 - b], s_in[1 - b])
            cp_in[b].wait()
            if r >= 2:
                cp_out[b].wait()
            compute(b)
            cp_out[b] = pltpu.async_copy(
                o_v[b], out_h.at[pl.ds(base_of(r), C)], s_out[b])

        for r in (FULL - 2, FULL - 1):
            if r >= 0:
                cp_out[r % 2].wait()

        if TAIL:
            @pl.when(wid < TAIL)
            def _():
                base = (FULL * NW + wid) * C
                pltpu.sync_copy(batch_h.at[pl.ds(base, C)], idx_v[0])
                compute(0)
                pltpu.sync_copy(o_v[0], out_h.at[pl.ds(base, C)])

    return k


def kernel(batch, time, forward_data_schedule, forward_noise_schedule):
    N = batch.shape[0]
    B = time.shape[0]
    T = forward_data_schedule.shape[0]
    C = 4000
    while N % C or C % _LANES:
        C //= 2
    k = _build(N, B, T, C)
    d = k(batch, time, forward_data_schedule)
    n = k(batch, time, forward_noise_schedule)
    return jnp.reshape(d, (N, 1)), jnp.reshape(n, (N, 1))
